# Initial kernel scaffold; baseline (speedup 1.0000x reference)
import jax
import jax.numpy as jnp
from jax.experimental import pallas as pl


def kernel(node_features, edge_index, edge_type, W1, a1, W2, a2, gamma, beta):
    raise NotImplementedError("write your pallas kernel here")


# speedup vs baseline: 245.9995x; 245.9995x over previous
import functools

import jax
import jax.numpy as jnp
from jax import lax
from jax.experimental import pallas as pl
from jax.experimental.pallas import tpu as pltpu
from jax.experimental.pallas import tpu_sc as plsc

N = 10000
NP = 10240
R = 8
H = 128
E = 320000
EP = 327680
NC = 2
NS = 16
NW = NC * NS
EPW = EP // NW
SG = 128
NGRP = EPW // SG
ROWS_PER_TILE = NP // NS
BN = 1024


def _transform_body(x_ref, w_ref, a_ref, h_ref, sd_ref, ss_ref):
    x = x_ref[...]
    for r in range(R):
        h = jnp.dot(x, w_ref[r], preferred_element_type=jnp.float32)
        h_ref[:, r, :] = h
        a = a_ref[r]
        sd_ref[:, r] = jnp.dot(h, a[:H], preferred_element_type=jnp.float32)
        ss_ref[:, r] = jnp.dot(h, a[H:], preferred_element_type=jnp.float32)


def _transform(xp, W, a):
    h, sd, ss = pl.pallas_call(
        _transform_body,
        grid=(NP // BN,),
        in_specs=[
            pl.BlockSpec((BN, H), lambda nb: (nb, 0)),
            pl.BlockSpec((R, H, H), lambda nb: (0, 0, 0)),
            pl.BlockSpec((R, 2 * H), lambda nb: (0, 0)),
        ],
        out_specs=[
            pl.BlockSpec((BN, R, H), lambda nb: (nb, 0, 0)),
            pl.BlockSpec((BN, R), lambda nb: (nb, 0)),
            pl.BlockSpec((BN, R), lambda nb: (nb, 0)),
        ],
        out_shape=[
            jax.ShapeDtypeStruct((NP, R, H), jnp.float32),
            jax.ShapeDtypeStruct((NP, R), jnp.float32),
            jax.ShapeDtypeStruct((NP, R), jnp.float32),
        ],
    )(xp, W, a)
    return h.reshape(NP * R, H), sd.reshape(NP * R), ss.reshape(NP * R)


CH = 1280
SPC = CH // SG
NCH = EPW // CH


def _edge_body(src_hbm, dst_hbm, et_hbm, sd_hbm, ss_hbm, h_hbm,
               out_hbm, den_hbm,
               srcB, dstB, etB,
               dstbA, idxsA, idxdA, sdbA, ssbA, rowsA,
               dstbB, idxsB, idxdB, sdbB, ssbB, rowsB,
               eb, denb, semA, semB,
               out_sh, den_sh):
    cid = lax.axis_index("c")
    sid = lax.axis_index("s")
    wid = sid * NC + cid
    base = wid * EPW

    def _zero_row(i, _):
        for j in range(H // 16):
            rowsA[i, pl.ds(j * 16, 16)] = jnp.zeros((16,), jnp.float32)
        return 0
    lax.fori_loop(0, SG, _zero_row, 0)

    def _zero_den(i, _):
        denb[pl.ds(i * 16, 16)] = jnp.zeros((16,), jnp.float32)
        return 0
    lax.fori_loop(0, ROWS_PER_TILE // 16, _zero_den, 0)

    row0 = sid * ROWS_PER_TILE
    for p in range(ROWS_PER_TILE // SG):
        pltpu.sync_copy(rowsA, out_sh.at[pl.ds(row0 + p * SG, SG), :])
    pltpu.sync_copy(denb, den_sh.at[pl.ds(row0, ROWS_PER_TILE)])
    plsc.subcore_barrier()

    def _stage(q, dstb, idxs, idxd, sdb, ssb, rows, sem):
        qo = q * SG
        for i in range(SG // 16):
            s16 = srcB[pl.ds(qo + i * 16, 16)]
            d16 = dstB[pl.ds(qo + i * 16, 16)]
            e16 = etB[pl.ds(qo + i * 16, 16)]
            idxs[pl.ds(i * 16, 16)] = s16 * R + e16
            idxd[pl.ds(i * 16, 16)] = d16 * R + e16
            dstb[pl.ds(i * 16, 16)] = d16
        pltpu.async_copy(sd_hbm.at[idxd], sdb, sem)
        pltpu.async_copy(ss_hbm.at[idxs], ssb, sem)
        pltpu.async_copy(h_hbm.at[idxs], rows, sem)

    def _proc(dstb, idxs, idxd, sdb, ssb, rows, sem):
        pltpu.make_async_copy(sd_hbm.at[idxd], sdb, sem).wait()
        pltpu.make_async_copy(ss_hbm.at[idxs], ssb, sem).wait()
        pltpu.make_async_copy(h_hbm.at[idxs], rows, sem).wait()
        for i in range(SG // 16):
            t = sdb[pl.ds(i * 16, 16)] + ssb[pl.ds(i * 16, 16)]
            t = jnp.where(t >= 0.0, t, 0.2 * t)
            eb[pl.ds(i * 16, 16)] = jnp.exp(t)
        pltpu.sync_copy(eb, den_sh.at[dstb], add=True)

        def _scale(i, _):
            ev = eb[pl.ds(i * 16, 16)]
            for c in range(16):
                a = ev[c]
                row = rows.at[i * 16 + c]
                for j in range(H // 16):
                    row[pl.ds(j * 16, 16)] = row[pl.ds(j * 16, 16)] * a
            return 0
        lax.fori_loop(0, SG // 16, _scale, 0)
        pltpu.sync_copy(rows, out_sh.at[dstb], add=True)

    A = (dstbA, idxsA, idxdA, sdbA, ssbA, rowsA, semA)
    B = (dstbB, idxsB, idxdB, sdbB, ssbB, rowsB, semB)

    def _chunk(ci, _):
        off = base + ci * CH
        pltpu.sync_copy(src_hbm.at[pl.ds(off, CH)], srcB)
        pltpu.sync_copy(dst_hbm.at[pl.ds(off, CH)], dstB)
        pltpu.sync_copy(et_hbm.at[pl.ds(off, CH)], etB)
        _stage(0, *A)

        def _pair(it, _):
            _stage(2 * it + 1, *B)
            _proc(*A)
            _stage(2 * it + 2, *A)
            _proc(*B)
            return 0
        lax.fori_loop(0, SPC // 2 - 1, _pair, 0)
        _stage(SPC - 1, *B)
        _proc(*A)
        _proc(*B)
        return 0

    lax.fori_loop(0, NCH, _chunk, 0)
    plsc.subcore_barrier()

    for p in range(ROWS_PER_TILE // SG):
        r0 = row0 + p * SG
        pltpu.sync_copy(out_sh.at[pl.ds(r0, SG), :], rowsA)
        pltpu.sync_copy(rowsA, out_hbm.at[cid, pl.ds(r0, SG), :])
    pltpu.sync_copy(den_sh.at[pl.ds(row0, ROWS_PER_TILE)], denb)
    pltpu.sync_copy(denb, den_hbm.at[cid, pl.ds(row0, ROWS_PER_TILE)])


def _edge_pass(src_p, dst_p, et_p, sd, ss, h):
    mesh = plsc.VectorSubcoreMesh(core_axis_name="c", subcore_axis_name="s")
    f = functools.partial(
        pl.kernel,
        out_type=[
            jax.ShapeDtypeStruct((NC, NP, H), jnp.float32),
            jax.ShapeDtypeStruct((NC, NP), jnp.float32),
        ],
        mesh=mesh,
        scratch_types=[
            pltpu.VMEM((CH,), jnp.int32),
            pltpu.VMEM((CH,), jnp.int32),
            pltpu.VMEM((CH,), jnp.int32),
            pltpu.VMEM((SG,), jnp.int32),
            pltpu.VMEM((SG,), jnp.int32),
            pltpu.VMEM((SG,), jnp.int32),
            pltpu.VMEM((SG,), jnp.float32),
            pltpu.VMEM((SG,), jnp.float32),
            pltpu.VMEM((SG, H), jnp.float32),
            pltpu.VMEM((SG,), jnp.int32),
            pltpu.VMEM((SG,), jnp.int32),
            pltpu.VMEM((SG,), jnp.int32),
            pltpu.VMEM((SG,), jnp.float32),
            pltpu.VMEM((SG,), jnp.float32),
            pltpu.VMEM((SG, H), jnp.float32),
            pltpu.VMEM((SG,), jnp.float32),
            pltpu.VMEM((ROWS_PER_TILE,), jnp.float32),
            pltpu.SemaphoreType.DMA,
            pltpu.SemaphoreType.DMA,
            pltpu.VMEM_SHARED((NP, H), jnp.float32),
            pltpu.VMEM_SHARED((NP,), jnp.float32),
        ],
    )(_edge_body)
    return f(src_p, dst_p, et_p, sd, ss, h)


def _relu_combine_body(out_ref, den_ref, y_ref):
    acc = out_ref[0] + out_ref[1]
    den = den_ref[0] + den_ref[1]
    y = jnp.where(den[:, None] > 0.0, acc / den[:, None], 0.0)
    y_ref[...] = jnp.maximum(y, 0.0)


def _relu_combine(out_parts, den_parts):
    return pl.pallas_call(
        _relu_combine_body,
        grid=(NP // BN,),
        in_specs=[
            pl.BlockSpec((NC, BN, H), lambda nb: (0, nb, 0)),
            pl.BlockSpec((NC, BN), lambda nb: (0, nb)),
        ],
        out_specs=pl.BlockSpec((BN, H), lambda nb: (nb, 0)),
        out_shape=jax.ShapeDtypeStruct((NP, H), jnp.float32),
    )(out_parts, den_parts)


def _bn_combine_body(out_ref, den_ref, gamma_ref, beta_ref, y_ref):
    acc = out_ref[0] + out_ref[1]
    den = den_ref[0] + den_ref[1]
    x = jnp.where(den[:, None] > 0.0, acc / den[:, None], 0.0)[:N]
    mean = jnp.mean(x, axis=0)
    var = jnp.mean((x - mean[None, :]) ** 2, axis=0)
    xh = (x - mean[None, :]) * jax.lax.rsqrt(var[None, :] + 1e-5)
    xh = xh * gamma_ref[...][None, :] + beta_ref[...][None, :]
    y_ref[...] = jnp.where(xh >= 0.0, xh, 0.01 * xh)


def _bn_combine(out_parts, den_parts, gamma, beta):
    return pl.pallas_call(
        _bn_combine_body,
        out_shape=jax.ShapeDtypeStruct((N, H), jnp.float32),
    )(out_parts, den_parts, gamma, beta)


def kernel(node_features, edge_index, edge_type, W1, a1, W2, a2, gamma, beta):
    xp = jnp.pad(node_features, ((0, NP - N), (0, 0)))
    pad = EP - E
    src_p = jnp.concatenate([edge_index[0], jnp.full((pad,), NP - 1, jnp.int32)])
    dst_p = jnp.concatenate([edge_index[1], jnp.full((pad,), NP - 1, jnp.int32)])
    et_p = jnp.concatenate([edge_type, jnp.zeros((pad,), jnp.int32)])

    h1, sd1, ss1 = _transform(xp, W1, a1)
    out1, den1 = _edge_pass(src_p, dst_p, et_p, sd1, ss1, h1)
    x2 = _relu_combine(out1, den1)

    h2, sd2, ss2 = _transform(x2, W2, a2)
    out2, den2 = _edge_pass(src_p, dst_p, et_p, sd2, ss2, h2)
    return _bn_combine(out2, den2, gamma, beta)


# speedup vs baseline: 588.8236x; 2.3936x over previous
import functools

import jax
import jax.numpy as jnp
from jax import lax
from jax.experimental import pallas as pl
from jax.experimental.pallas import tpu as pltpu
from jax.experimental.pallas import tpu_sc as plsc

N = 10000
NP = 10240
R = 8
H = 128
E = 320000
EP = 327680
NC = 2
NS = 16
NW = NC * NS
EPW = EP // NW
SG = 128
NGRP = EPW // SG
ROWS_PER_TILE = NP // NS
BN = 1024


def _transform_body(x_ref, w_ref, a_ref, h_ref, sd_ref, ss_ref):
    x = x_ref[...]
    for r in range(R):
        h = jnp.dot(x, w_ref[r], preferred_element_type=jnp.float32)
        h_ref[:, r, :] = h
        a = a_ref[r]
        sd_ref[:, r] = jnp.dot(h, a[:H], preferred_element_type=jnp.float32)
        ss_ref[:, r] = jnp.dot(h, a[H:], preferred_element_type=jnp.float32)


def _transform(xp, W, a):
    h, sd, ss = pl.pallas_call(
        _transform_body,
        grid=(NP // BN,),
        in_specs=[
            pl.BlockSpec((BN, H), lambda nb: (nb, 0)),
            pl.BlockSpec((R, H, H), lambda nb: (0, 0, 0)),
            pl.BlockSpec((R, 2 * H), lambda nb: (0, 0)),
        ],
        out_specs=[
            pl.BlockSpec((BN, R, H), lambda nb: (nb, 0, 0)),
            pl.BlockSpec((BN, R), lambda nb: (nb, 0)),
            pl.BlockSpec((BN, R), lambda nb: (nb, 0)),
        ],
        out_shape=[
            jax.ShapeDtypeStruct((NP, R, H), jnp.float32),
            jax.ShapeDtypeStruct((NP, R), jnp.float32),
            jax.ShapeDtypeStruct((NP, R), jnp.float32),
        ],
    )(xp, W, a)
    return h.reshape(NP * R, H), sd.reshape(NP * R), ss.reshape(NP * R)


CH = 1280
SPC = CH // SG
NCH = EPW // CH


def _edge_body(src_hbm, dst_hbm, et_hbm, sd_hbm, ss_hbm, h_hbm,
               out_hbm, den_hbm,
               srcB, dstB, etB,
               dstbA, idxsA, idxdA, sdbA, ssbA, rowsA,
               dstbB, idxsB, idxdB, sdbB, ssbB, rowsB,
               eb, denb, semA, semB,
               out_sh, den_sh):
    cid = lax.axis_index("c")
    sid = lax.axis_index("s")
    wid = sid * NC + cid
    base = wid * EPW

    def _zero_row(i, _):
        for j in range(H // 16):
            rowsA[i, pl.ds(j * 16, 16)] = jnp.zeros((16,), jnp.float32)
        return 0
    lax.fori_loop(0, SG, _zero_row, 0)

    def _zero_den(i, _):
        denb[pl.ds(i * 16, 16)] = jnp.zeros((16,), jnp.float32)
        return 0
    lax.fori_loop(0, ROWS_PER_TILE // 16, _zero_den, 0)

    row0 = sid * ROWS_PER_TILE
    for p in range(ROWS_PER_TILE // SG):
        pltpu.sync_copy(rowsA, out_sh.at[pl.ds(row0 + p * SG, SG), :])
    pltpu.sync_copy(denb, den_sh.at[pl.ds(row0, ROWS_PER_TILE)])
    plsc.subcore_barrier()

    def _stage(q, dstb, idxs, idxd, sdb, ssb, rows, sem):
        qo = q * SG
        for i in range(SG // 16):
            s16 = srcB[pl.ds(qo + i * 16, 16)]
            d16 = dstB[pl.ds(qo + i * 16, 16)]
            e16 = etB[pl.ds(qo + i * 16, 16)]
            idxs[pl.ds(i * 16, 16)] = s16 * R + e16
            idxd[pl.ds(i * 16, 16)] = d16 * R + e16
            dstb[pl.ds(i * 16, 16)] = d16
        pltpu.async_copy(sd_hbm.at[idxd], sdb, sem)
        pltpu.async_copy(ss_hbm.at[idxs], ssb, sem)
        pltpu.async_copy(h_hbm.at[idxs], rows, sem)

    def _proc(dstb, idxs, idxd, sdb, ssb, rows, sem):
        pltpu.make_async_copy(sd_hbm.at[idxd], sdb, sem).wait()
        pltpu.make_async_copy(ss_hbm.at[idxs], ssb, sem).wait()
        pltpu.make_async_copy(h_hbm.at[idxs], rows, sem).wait()
        for i in range(SG // 16):
            t = sdb[pl.ds(i * 16, 16)] + ssb[pl.ds(i * 16, 16)]
            t = jnp.where(t >= 0.0, t, 0.2 * t)
            eb[pl.ds(i * 16, 16)] = jnp.exp(t)
        pltpu.sync_copy(eb, den_sh.at[dstb], add=True)

        def _scale(i, _):
            ev = eb[pl.ds(i * 16, 16)]
            for c in range(16):
                a = ev[c]
                row = rows.at[i * 16 + c]
                for j in range(H // 16):
                    row[pl.ds(j * 16, 16)] = row[pl.ds(j * 16, 16)] * a
            return 0
        lax.fori_loop(0, SG // 16, _scale, 0)
        pltpu.sync_copy(rows, out_sh.at[dstb], add=True)

    A = (dstbA, idxsA, idxdA, sdbA, ssbA, rowsA, semA)
    B = (dstbB, idxsB, idxdB, sdbB, ssbB, rowsB, semB)

    def _chunk(ci, _):
        off = base + ci * CH
        pltpu.sync_copy(src_hbm.at[pl.ds(off, CH)], srcB)
        pltpu.sync_copy(dst_hbm.at[pl.ds(off, CH)], dstB)
        pltpu.sync_copy(et_hbm.at[pl.ds(off, CH)], etB)
        _stage(0, *A)

        def _pair(it, _):
            _stage(2 * it + 1, *B)
            _proc(*A)
            _stage(2 * it + 2, *A)
            _proc(*B)
            return 0
        lax.fori_loop(0, SPC // 2 - 1, _pair, 0)
        _stage(SPC - 1, *B)
        _proc(*A)
        _proc(*B)
        return 0

    lax.fori_loop(0, NCH, _chunk, 0)
    plsc.subcore_barrier()

    for p in range(ROWS_PER_TILE // SG):
        r0 = row0 + p * SG
        pltpu.sync_copy(out_sh.at[pl.ds(r0, SG), :], rowsA)
        pltpu.sync_copy(rowsA, out_hbm.at[cid, pl.ds(r0, SG), :])
    pltpu.sync_copy(den_sh.at[pl.ds(row0, ROWS_PER_TILE)], denb)
    pltpu.sync_copy(denb, den_hbm.at[cid, pl.ds(row0, ROWS_PER_TILE)])


def _edge_pass(src_p, dst_p, et_p, sd, ss, h):
    mesh = plsc.VectorSubcoreMesh(core_axis_name="c", subcore_axis_name="s")
    f = functools.partial(
        pl.kernel,
        out_type=[
            jax.ShapeDtypeStruct((NC, NP, H), jnp.float32),
            jax.ShapeDtypeStruct((NC, NP), jnp.float32),
        ],
        mesh=mesh,
        scratch_types=[
            pltpu.VMEM((CH,), jnp.int32),
            pltpu.VMEM((CH,), jnp.int32),
            pltpu.VMEM((CH,), jnp.int32),
            pltpu.VMEM((SG,), jnp.int32),
            pltpu.VMEM((SG,), jnp.int32),
            pltpu.VMEM((SG,), jnp.int32),
            pltpu.VMEM((SG,), jnp.float32),
            pltpu.VMEM((SG,), jnp.float32),
            pltpu.VMEM((SG, H), jnp.float32),
            pltpu.VMEM((SG,), jnp.int32),
            pltpu.VMEM((SG,), jnp.int32),
            pltpu.VMEM((SG,), jnp.int32),
            pltpu.VMEM((SG,), jnp.float32),
            pltpu.VMEM((SG,), jnp.float32),
            pltpu.VMEM((SG, H), jnp.float32),
            pltpu.VMEM((SG,), jnp.float32),
            pltpu.VMEM((ROWS_PER_TILE,), jnp.float32),
            pltpu.SemaphoreType.DMA,
            pltpu.SemaphoreType.DMA,
            pltpu.VMEM_SHARED((NP, H), jnp.float32),
            pltpu.VMEM_SHARED((NP,), jnp.float32),
        ],
    )(_edge_body)
    return f(src_p, dst_p, et_p, sd, ss, h)


def _relu_combine_body(out_ref, den_ref, y_ref):
    acc = out_ref[0] + out_ref[1]
    den = den_ref[0] + den_ref[1]
    y = jnp.where(den[:, None] > 0.0, acc / den[:, None], 0.0)
    y_ref[...] = jnp.maximum(y, 0.0)


def _relu_combine(out_parts, den_parts):
    return pl.pallas_call(
        _relu_combine_body,
        grid=(NP // BN,),
        in_specs=[
            pl.BlockSpec((NC, BN, H), lambda nb: (0, nb, 0)),
            pl.BlockSpec((NC, BN), lambda nb: (0, nb)),
        ],
        out_specs=pl.BlockSpec((BN, H), lambda nb: (nb, 0)),
        out_shape=jax.ShapeDtypeStruct((NP, H), jnp.float32),
    )(out_parts, den_parts)


def _bn_combine_body(out_ref, den_ref, gamma_ref, beta_ref, y_ref):
    acc = out_ref[0] + out_ref[1]
    den = den_ref[0] + den_ref[1]
    x = jnp.where(den[:, None] > 0.0, acc / den[:, None], 0.0)[:N]
    mean = jnp.mean(x, axis=0)
    var = jnp.mean((x - mean[None, :]) ** 2, axis=0)
    xh = (x - mean[None, :]) * jax.lax.rsqrt(var[None, :] + 1e-5)
    xh = xh * gamma_ref[...][None, :] + beta_ref[...][None, :]
    y_ref[...] = jnp.where(xh >= 0.0, xh, 0.01 * xh)


def _bn_combine(out_parts, den_parts, gamma, beta):
    return pl.pallas_call(
        _bn_combine_body,
        out_shape=jax.ShapeDtypeStruct((N, H), jnp.float32),
    )(out_parts, den_parts, gamma, beta)


def kernel(node_features, edge_index, edge_type, W1, a1, W2, a2, gamma, beta):
    xp = jnp.pad(node_features, ((0, NP - N), (0, 0)))
    pad = EP - E
    dummy = N + (jnp.arange(pad, dtype=jnp.int32) % (NP - N))
    src_p = jnp.concatenate([edge_index[0], dummy])
    dst_p = jnp.concatenate([edge_index[1], dummy])
    et_p = jnp.concatenate([edge_type, jnp.zeros((pad,), jnp.int32)])

    h1, sd1, ss1 = _transform(xp, W1, a1)
    out1, den1 = _edge_pass(src_p, dst_p, et_p, sd1, ss1, h1)
    x2 = _relu_combine(out1, den1)

    h2, sd2, ss2 = _transform(x2, W2, a2)
    out2, den2 = _edge_pass(src_p, dst_p, et_p, sd2, ss2, h2)
    return _bn_combine(out2, den2, gamma, beta)
